# baseline (speedup 1.0000x reference)
import functools

import jax
import jax.numpy as jnp
from jax.experimental import pallas as pl
from jax.experimental.pallas import tpu as pltpu

_B_BLOCK = 1024
_K = 1024
_DIM = 256
_NCB = 4


def _rvq_body(q_ref, cbs_ref, idx_ref, emb_ref):
    q = q_ref[...]
    r = q
    col = jax.lax.broadcasted_iota(jnp.int32, (q.shape[0], _K), 1)
    for i in range(_NCB):
        cb = cbs_ref[i]
        sims = jax.lax.dot_general(
            r, cb, (((1,), (1,)), ((), ())),
            preferred_element_type=jnp.float32)
        m = jnp.max(sims, axis=1, keepdims=True)
        idx = jnp.min(jnp.where(sims == m, col, _K), axis=1).astype(jnp.int32)
        onehot = (col == idx[:, None]).astype(jnp.float32)
        sel = jax.lax.dot_general(
            onehot, cb, (((1,), (0,)), ((), ())),
            precision=jax.lax.Precision.HIGHEST,
            preferred_element_type=jnp.float32)
        r = r - sel
        idx_ref[i, :] = idx
    emb_ref[...] = q - r


@jax.jit
def kernel(query, cb0, cb1, cb2, cb3):
    B = query.shape[0]
    cbs = jnp.stack([cb0, cb1, cb2, cb3], axis=0)
    grid = (B // _B_BLOCK,)
    idx, emb = pl.pallas_call(
        _rvq_body,
        grid=grid,
        in_specs=[
            pl.BlockSpec((_B_BLOCK, _DIM), lambda i: (i, 0)),
            pl.BlockSpec((_NCB, _K, _DIM), lambda i: (0, 0, 0)),
        ],
        out_specs=[
            pl.BlockSpec((_NCB, _B_BLOCK), lambda i: (0, i)),
            pl.BlockSpec((_B_BLOCK, _DIM), lambda i: (i, 0)),
        ],
        out_shape=[
            jax.ShapeDtypeStruct((_NCB, B), jnp.int32),
            jax.ShapeDtypeStruct((B, _DIM), jnp.float32),
        ],
        compiler_params=pltpu.CompilerParams(
            dimension_semantics=("arbitrary",),
        ),
    )(query, cbs)
    return idx, emb


# speedup vs baseline: 1.6484x; 1.6484x over previous
import jax
import jax.numpy as jnp
from jax.experimental import pallas as pl
from jax.experimental.pallas import tpu as pltpu

_B_BLOCK = 1024
_K = 1024
_DIM = 256
_NCB = 4


def _rvq_body(q_ref, hi_ref, mid_ref, lo_ref, idx_ref, emb_ref):
    q = q_ref[...]
    r = q
    col = jax.lax.broadcasted_iota(jnp.int32, (q.shape[0], _K), 1)
    for i in range(_NCB):
        hi = hi_ref[i]
        sims = jax.lax.dot_general(
            r.astype(jnp.bfloat16), hi, (((1,), (1,)), ((), ())),
            preferred_element_type=jnp.float32)
        m = jnp.max(sims, axis=1, keepdims=True)
        idx = jnp.min(jnp.where(sims == m, col, _K), axis=1).astype(jnp.int32)
        onehot = (col == idx[:, None]).astype(jnp.bfloat16)
        sel = jax.lax.dot_general(
            onehot, hi, (((1,), (0,)), ((), ())),
            preferred_element_type=jnp.float32)
        sel += jax.lax.dot_general(
            onehot, mid_ref[i], (((1,), (0,)), ((), ())),
            preferred_element_type=jnp.float32)
        sel += jax.lax.dot_general(
            onehot, lo_ref[i], (((1,), (0,)), ((), ())),
            preferred_element_type=jnp.float32)
        r = r - sel
        idx_ref[i, :] = idx
    emb_ref[...] = q - r


@jax.jit
def kernel(query, cb0, cb1, cb2, cb3):
    B = query.shape[0]
    cbs = jnp.stack([cb0, cb1, cb2, cb3], axis=0)
    hi = cbs.astype(jnp.bfloat16)
    rem = cbs - hi.astype(jnp.float32)
    mid = rem.astype(jnp.bfloat16)
    lo = (rem - mid.astype(jnp.float32)).astype(jnp.bfloat16)
    grid = (B // _B_BLOCK,)
    cb_spec = pl.BlockSpec((_NCB, _K, _DIM), lambda i: (0, 0, 0))
    idx, emb = pl.pallas_call(
        _rvq_body,
        grid=grid,
        in_specs=[
            pl.BlockSpec((_B_BLOCK, _DIM), lambda i: (i, 0)),
            cb_spec, cb_spec, cb_spec,
        ],
        out_specs=[
            pl.BlockSpec((_NCB, _B_BLOCK), lambda i: (0, i)),
            pl.BlockSpec((_B_BLOCK, _DIM), lambda i: (i, 0)),
        ],
        out_shape=[
            jax.ShapeDtypeStruct((_NCB, B), jnp.int32),
            jax.ShapeDtypeStruct((B, _DIM), jnp.float32),
        ],
        compiler_params=pltpu.CompilerParams(
            dimension_semantics=("arbitrary",),
        ),
    )(query, hi, mid, lo)
    return idx, emb
